# baseline (speedup 1.0000x reference)
import functools
import jax
import jax.numpy as jnp
from jax import lax
from jax.experimental import pallas as pl
from jax.experimental.pallas import tpu as pltpu
from jax.experimental.pallas import tpu_sc as plsc

N = 10000
E = 320000
H = 128
G = 64
C = 10

NC = 2
NS = 16
NW = NC * NS
K = 128
CH = 80
NCH = NW * CH
EPAD = NCH * K
CHA = 80
CHB = 160 - CHA

NPAD = 10240
RPT = NPAD // NS
HRPT = RPT // 2
NB = 8
R = NPAD // NB

_mesh = plsc.VectorSubcoreMesh(core_axis_name="c", subcore_axis_name="s")


@functools.partial(
    pl.kernel,
    out_type=jax.ShapeDtypeStruct((NC, NPAD, H), jnp.float32),
    mesh=_mesh,
    scratch_types=[
        pltpu.VMEM((CH, K), jnp.int32),
        pltpu.VMEM((K, H), jnp.float32),
        pltpu.VMEM_SHARED((NPAD, H), jnp.float32),
    ],
)
def _deg_kernel(dst_hbm, ones_hbm, z_hbm, deg_out, dst_v, ones_v, acc):
    c = lax.axis_index("c")
    s = lax.axis_index("s")
    w = c * NS + s
    pltpu.sync_copy(dst_hbm.at[pl.ds(w * CH, CH)], dst_v)
    pltpu.sync_copy(ones_hbm, ones_v)
    pltpu.sync_copy(z_hbm, acc.at[pl.ds(s * RPT, RPT)])
    plsc.subcore_barrier()

    def body(j, carry):
        pltpu.sync_copy(ones_v, acc.at[dst_v.at[j]], add=True)
        return carry

    lax.fori_loop(0, CH, body, 0)
    plsc.subcore_barrier()
    pltpu.sync_copy(acc.at[pl.ds(s * RPT, RPT)], deg_out.at[c, pl.ds(s * RPT, RPT)])


W = 16


@functools.partial(
    pl.kernel,
    out_type=jax.ShapeDtypeStruct((NC, NPAD, H), jnp.float32),
    mesh=_mesh,
    scratch_types=[
        pltpu.VMEM((2, W, K), jnp.int32),
        pltpu.VMEM((2, W, K), jnp.int32),
        pltpu.VMEM((2, K, H), jnp.float32),
        pltpu.SemaphoreType.DMA,
        pltpu.SemaphoreType.DMA,
        pltpu.VMEM_SHARED((NPAD, H), jnp.float32),
    ],
)
def _mp_kernel(g_hbm, src_hbm, dst_hbm, z_hbm, m_out,
               src_w, dst_w, rows, sem0, sem1, acc):
    c = lax.axis_index("c")
    s = lax.axis_index("s")
    sems = (sem0, sem1)
    ch = jnp.where(c == 0, CHA, CHB)
    nwin = ch // W
    base = jnp.where(c == 0, s * CHA, NS * CHA + s * CHB)
    pltpu.sync_copy(src_hbm.at[pl.ds(base, W)], src_w.at[0])
    pltpu.sync_copy(dst_hbm.at[pl.ds(base, W)], dst_w.at[0])
    pltpu.sync_copy(z_hbm, acc.at[pl.ds(s * RPT, RPT)])
    plsc.subcore_barrier()

    @pl.when(ch > 0)
    def _():
        pltpu.async_copy(g_hbm.at[src_w.at[0, 0]], rows.at[0], sems[0])

    def win(v, carry):
        v2 = lax.rem(v, 2)
        v2n = lax.rem(v + 1, 2)

        @pl.when(v + 1 < nwin)
        def _():
            pltpu.sync_copy(src_hbm.at[pl.ds(base + (v + 1) * W, W)],
                            src_w.at[v2n])
            pltpu.sync_copy(dst_hbm.at[pl.ds(base + (v + 1) * W, W)],
                            dst_w.at[v2n])

        for t in range(W):
            j = v * W + t
            b, bn = t % 2, (t + 1) % 2
            pltpu.make_async_copy(
                g_hbm.at[src_w.at[v2, t]], rows.at[b], sems[b]).wait()
            nxt = (src_w.at[v2, t + 1] if t + 1 < W
                   else src_w.at[v2n, 0])

            @pl.when(j + 1 < ch)
            def _():
                pltpu.async_copy(g_hbm.at[nxt], rows.at[bn], sems[bn])

            pltpu.sync_copy(rows.at[b], acc.at[dst_w.at[v2, t]], add=True)
        return carry

    lax.fori_loop(0, nwin, win, 0)
    plsc.subcore_barrier()
    pltpu.sync_copy(acc.at[pl.ds(s * RPT, RPT)], m_out.at[c, pl.ds(s * RPT, RPT)])


def _pre_body(deg_ref, x_ref, w_ref, g_ref, dinv_ref):
    deg = deg_ref[0] + deg_ref[1] + 1.0
    dinvb = lax.rsqrt(deg)
    hw = jnp.dot(x_ref[...], w_ref[...], preferred_element_type=jnp.float32)
    g_ref[...] = dinvb * hw
    dinv_ref[...] = dinvb


_pre = pl.pallas_call(
    _pre_body,
    grid=(NB,),
    in_specs=[
        pl.BlockSpec((2, R, H), lambda i: (0, i, 0)),
        pl.BlockSpec((R, H), lambda i: (i, 0)),
        pl.BlockSpec((H, H), lambda i: (0, 0)),
    ],
    out_specs=[
        pl.BlockSpec((R, H), lambda i: (i, 0)),
        pl.BlockSpec((R, H), lambda i: (i, 0)),
    ],
    out_shape=[
        jax.ShapeDtypeStruct((NPAD, H), jnp.float32),
        jax.ShapeDtypeStruct((NPAD, H), jnp.float32),
    ],
)


def _mid_body(m_ref, g_ref, dinv_ref, b_ref, w_ref, o_ref):
    t = (m_ref[0] + m_ref[1] + g_ref[...]) * dinv_ref[...] + b_ref[...]
    h = jnp.maximum(t, 0.0)
    o_ref[...] = dinv_ref[...] * jnp.dot(
        h, w_ref[...], preferred_element_type=jnp.float32)


_mid = pl.pallas_call(
    _mid_body,
    grid=(NB,),
    in_specs=[
        pl.BlockSpec((2, R, H), lambda i: (0, i, 0)),
        pl.BlockSpec((R, H), lambda i: (i, 0)),
        pl.BlockSpec((R, H), lambda i: (i, 0)),
        pl.BlockSpec((1, H), lambda i: (0, 0)),
        pl.BlockSpec((H, H), lambda i: (0, 0)),
    ],
    out_specs=pl.BlockSpec((R, H), lambda i: (i, 0)),
    out_shape=jax.ShapeDtypeStruct((NPAD, H), jnp.float32),
)


def _final_body(m_ref, g_ref, dinv_ref, b_ref, bt_ref, wl_ref, bl_ref,
                o_ref, acc, cnt):
    i = pl.program_id(0)

    @pl.when(i == 0)
    def _():
        acc[...] = jnp.zeros_like(acc)
        cnt[...] = jnp.zeros_like(cnt)

    h = (m_ref[0] + m_ref[1] + g_ref[...]) * dinv_ref[...] + b_ref[...]
    bt = bt_ref[0]
    P = (lax.broadcasted_iota(jnp.int32, (G, R), 0) == bt
         ).astype(jnp.float32)
    acc[...] += jnp.dot(P, h, preferred_element_type=jnp.float32)
    cnt[...] += jnp.dot(P, jnp.ones((R, H), jnp.float32),
                        preferred_element_type=jnp.float32)

    @pl.when(i == NB - 1)
    def _():
        pooled = acc[...] / jnp.maximum(cnt[...], 1.0)
        o_ref[...] = jnp.dot(pooled, wl_ref[...],
                             preferred_element_type=jnp.float32) + bl_ref[...]


_final = pl.pallas_call(
    _final_body,
    grid=(NB,),
    in_specs=[
        pl.BlockSpec((2, R, H), lambda i: (0, i, 0)),
        pl.BlockSpec((R, H), lambda i: (i, 0)),
        pl.BlockSpec((R, H), lambda i: (i, 0)),
        pl.BlockSpec((1, H), lambda i: (0, 0)),
        pl.BlockSpec((1, 1, R), lambda i: (i, 0, 0)),
        pl.BlockSpec((H, H), lambda i: (0, 0)),
        pl.BlockSpec((1, H), lambda i: (0, 0)),
    ],
    out_specs=pl.BlockSpec((G, H), lambda i: (0, 0)),
    out_shape=jax.ShapeDtypeStruct((G, H), jnp.float32),
    scratch_shapes=[
        pltpu.VMEM((G, H), jnp.float32),
        pltpu.VMEM((G, H), jnp.float32),
    ],
)


def kernel(x, edge_index, batch, W1, b1, W2, b2, W3, b3, Wl, bl):
    perm = jnp.argsort(edge_index[0])
    src = edge_index[0][perm]
    dst = edge_index[1][perm]
    pad_e = EPAD - E
    src_r = jnp.concatenate(
        [src, jnp.zeros((pad_e,), jnp.int32)]).reshape(NCH, K)
    dst_r = jnp.concatenate(
        [dst, jnp.full((pad_e,), N, jnp.int32)]).reshape(NCH, K)

    x_pad = jnp.pad(x, ((0, NPAD - N), (0, 0)))
    batch_p = jnp.pad(batch, (0, NPAD - N),
                      constant_values=G).reshape(NB, 1, R)
    Wl_p = jnp.pad(Wl, ((0, 0), (0, H - C)))
    bl_p = jnp.pad(bl, (0, H - C)).reshape(1, H)

    ones128 = jnp.ones((K, H), jnp.float32)
    z128 = jnp.zeros((RPT, H), jnp.float32)

    deg_par = _deg_kernel(dst_r, ones128, z128)
    g1, dinvb = _pre(deg_par, x_pad, W1)
    m1 = _mp_kernel(g1, src_r, dst_r, z128)
    g2 = _mid(m1, g1, dinvb, b1.reshape(1, H), W2)
    m2 = _mp_kernel(g2, src_r, dst_r, z128)
    g3 = _mid(m2, g2, dinvb, b2.reshape(1, H), W3)
    m3 = _mp_kernel(g3, src_r, dst_r, z128)
    out = _final(m3, g3, dinvb, b3.reshape(1, H), batch_p, Wl_p, bl_p)
    return out[:, :C]


# speedup vs baseline: 1.2989x; 1.2989x over previous
import functools
import jax
import jax.numpy as jnp
from jax import lax
from jax.experimental import pallas as pl
from jax.experimental.pallas import tpu as pltpu
from jax.experimental.pallas import tpu_sc as plsc

N = 10000
E = 320000
H = 128
G = 64
C = 10

NC = 2
NS = 16
NW = NC * NS
K = 128
CH = 80
NCH = NW * CH
EPAD = NCH * K
CHA = 32
CHB = 160 - CHA

NPAD = 10240
RPT = NPAD // NS
HRPT = RPT // 2
NB = 8
R = NPAD // NB

_mesh = plsc.VectorSubcoreMesh(core_axis_name="c", subcore_axis_name="s")


@functools.partial(
    pl.kernel,
    out_type=jax.ShapeDtypeStruct((NC, NPAD, H), jnp.float32),
    mesh=_mesh,
    scratch_types=[
        pltpu.VMEM((CH, K), jnp.int32),
        pltpu.VMEM((K, H), jnp.float32),
        pltpu.VMEM_SHARED((NPAD, H), jnp.float32),
    ],
)
def _deg_kernel(dst_hbm, ones_hbm, z_hbm, deg_out, dst_v, ones_v, acc):
    c = lax.axis_index("c")
    s = lax.axis_index("s")
    w = c * NS + s
    pltpu.sync_copy(dst_hbm.at[pl.ds(w * CH, CH)], dst_v)
    pltpu.sync_copy(ones_hbm, ones_v)
    pltpu.sync_copy(z_hbm, acc.at[pl.ds(s * RPT, RPT)])
    plsc.subcore_barrier()

    def body(j, carry):
        pltpu.sync_copy(ones_v, acc.at[dst_v.at[j]], add=True)
        return carry

    lax.fori_loop(0, CH, body, 0)
    plsc.subcore_barrier()
    pltpu.sync_copy(acc.at[pl.ds(s * RPT, RPT)], deg_out.at[c, pl.ds(s * RPT, RPT)])


W = 16


@functools.partial(
    pl.kernel,
    out_type=jax.ShapeDtypeStruct((NC, NPAD, H), jnp.float32),
    mesh=_mesh,
    scratch_types=[
        pltpu.VMEM((2, W, K), jnp.int32),
        pltpu.VMEM((2, W, K), jnp.int32),
        pltpu.VMEM((2, K, H), jnp.float32),
        pltpu.SemaphoreType.DMA,
        pltpu.SemaphoreType.DMA,
        pltpu.VMEM_SHARED((NPAD, H), jnp.float32),
    ],
)
def _mp_kernel(g_hbm, src_hbm, dst_hbm, z_hbm, m_out,
               src_w, dst_w, rows, sem0, sem1, acc):
    c = lax.axis_index("c")
    s = lax.axis_index("s")
    sems = (sem0, sem1)
    ch = jnp.where(c == 0, CHA, CHB)
    nwin = ch // W
    base = jnp.where(c == 0, s * CHA, NS * CHA + s * CHB)
    pltpu.sync_copy(src_hbm.at[pl.ds(base, W)], src_w.at[0])
    pltpu.sync_copy(dst_hbm.at[pl.ds(base, W)], dst_w.at[0])
    pltpu.sync_copy(z_hbm, acc.at[pl.ds(s * RPT, RPT)])
    plsc.subcore_barrier()

    @pl.when(ch > 0)
    def _():
        pltpu.async_copy(g_hbm.at[src_w.at[0, 0]], rows.at[0], sems[0])

    def win(v, carry):
        v2 = lax.rem(v, 2)
        v2n = lax.rem(v + 1, 2)

        @pl.when(v + 1 < nwin)
        def _():
            pltpu.sync_copy(src_hbm.at[pl.ds(base + (v + 1) * W, W)],
                            src_w.at[v2n])
            pltpu.sync_copy(dst_hbm.at[pl.ds(base + (v + 1) * W, W)],
                            dst_w.at[v2n])

        for t in range(W):
            j = v * W + t
            b, bn = t % 2, (t + 1) % 2
            pltpu.make_async_copy(
                g_hbm.at[src_w.at[v2, t]], rows.at[b], sems[b]).wait()
            nxt = (src_w.at[v2, t + 1] if t + 1 < W
                   else src_w.at[v2n, 0])

            @pl.when(j + 1 < ch)
            def _():
                pltpu.async_copy(g_hbm.at[nxt], rows.at[bn], sems[bn])

            pltpu.sync_copy(rows.at[b], acc.at[dst_w.at[v2, t]], add=True)
        return carry

    lax.fori_loop(0, nwin, win, 0)
    plsc.subcore_barrier()
    pltpu.sync_copy(acc.at[pl.ds(s * RPT, RPT)], m_out.at[c, pl.ds(s * RPT, RPT)])


def _pre_body(deg_ref, x_ref, w_ref, g_ref, dinv_ref):
    deg = deg_ref[0] + deg_ref[1] + 1.0
    dinvb = lax.rsqrt(deg)
    hw = jnp.dot(x_ref[...], w_ref[...], preferred_element_type=jnp.float32)
    g_ref[...] = dinvb * hw
    dinv_ref[...] = dinvb


_pre = pl.pallas_call(
    _pre_body,
    grid=(NB,),
    in_specs=[
        pl.BlockSpec((2, R, H), lambda i: (0, i, 0)),
        pl.BlockSpec((R, H), lambda i: (i, 0)),
        pl.BlockSpec((H, H), lambda i: (0, 0)),
    ],
    out_specs=[
        pl.BlockSpec((R, H), lambda i: (i, 0)),
        pl.BlockSpec((R, H), lambda i: (i, 0)),
    ],
    out_shape=[
        jax.ShapeDtypeStruct((NPAD, H), jnp.float32),
        jax.ShapeDtypeStruct((NPAD, H), jnp.float32),
    ],
)


def _mid_body(m_ref, g_ref, dinv_ref, b_ref, w_ref, o_ref):
    t = (m_ref[0] + m_ref[1] + g_ref[...]) * dinv_ref[...] + b_ref[...]
    h = jnp.maximum(t, 0.0)
    o_ref[...] = dinv_ref[...] * jnp.dot(
        h, w_ref[...], preferred_element_type=jnp.float32)


_mid = pl.pallas_call(
    _mid_body,
    grid=(NB,),
    in_specs=[
        pl.BlockSpec((2, R, H), lambda i: (0, i, 0)),
        pl.BlockSpec((R, H), lambda i: (i, 0)),
        pl.BlockSpec((R, H), lambda i: (i, 0)),
        pl.BlockSpec((1, H), lambda i: (0, 0)),
        pl.BlockSpec((H, H), lambda i: (0, 0)),
    ],
    out_specs=pl.BlockSpec((R, H), lambda i: (i, 0)),
    out_shape=jax.ShapeDtypeStruct((NPAD, H), jnp.float32),
)


def _final_body(m_ref, g_ref, dinv_ref, b_ref, bt_ref, wl_ref, bl_ref,
                o_ref, acc, cnt):
    i = pl.program_id(0)

    @pl.when(i == 0)
    def _():
        acc[...] = jnp.zeros_like(acc)
        cnt[...] = jnp.zeros_like(cnt)

    h = (m_ref[0] + m_ref[1] + g_ref[...]) * dinv_ref[...] + b_ref[...]
    bt = bt_ref[0]
    P = (lax.broadcasted_iota(jnp.int32, (G, R), 0) == bt
         ).astype(jnp.float32)
    acc[...] += jnp.dot(P, h, preferred_element_type=jnp.float32)
    cnt[...] += jnp.dot(P, jnp.ones((R, H), jnp.float32),
                        preferred_element_type=jnp.float32)

    @pl.when(i == NB - 1)
    def _():
        pooled = acc[...] / jnp.maximum(cnt[...], 1.0)
        o_ref[...] = jnp.dot(pooled, wl_ref[...],
                             preferred_element_type=jnp.float32) + bl_ref[...]


_final = pl.pallas_call(
    _final_body,
    grid=(NB,),
    in_specs=[
        pl.BlockSpec((2, R, H), lambda i: (0, i, 0)),
        pl.BlockSpec((R, H), lambda i: (i, 0)),
        pl.BlockSpec((R, H), lambda i: (i, 0)),
        pl.BlockSpec((1, H), lambda i: (0, 0)),
        pl.BlockSpec((1, 1, R), lambda i: (i, 0, 0)),
        pl.BlockSpec((H, H), lambda i: (0, 0)),
        pl.BlockSpec((1, H), lambda i: (0, 0)),
    ],
    out_specs=pl.BlockSpec((G, H), lambda i: (0, 0)),
    out_shape=jax.ShapeDtypeStruct((G, H), jnp.float32),
    scratch_shapes=[
        pltpu.VMEM((G, H), jnp.float32),
        pltpu.VMEM((G, H), jnp.float32),
    ],
)


def kernel(x, edge_index, batch, W1, b1, W2, b2, W3, b3, Wl, bl):
    src = edge_index[0]
    dst = edge_index[1]
    pad_e = EPAD - E
    src_r = jnp.concatenate(
        [src, jnp.zeros((pad_e,), jnp.int32)]).reshape(NCH, K)
    dst_r = jnp.concatenate(
        [dst, jnp.full((pad_e,), N, jnp.int32)]).reshape(NCH, K)

    x_pad = jnp.pad(x, ((0, NPAD - N), (0, 0)))
    batch_p = jnp.pad(batch, (0, NPAD - N),
                      constant_values=G).reshape(NB, 1, R)
    Wl_p = jnp.pad(Wl, ((0, 0), (0, H - C)))
    bl_p = jnp.pad(bl, (0, H - C)).reshape(1, H)

    ones128 = jnp.ones((K, H), jnp.float32)
    z128 = jnp.zeros((RPT, H), jnp.float32)

    deg_par = _deg_kernel(dst_r, ones128, z128)
    g1, dinvb = _pre(deg_par, x_pad, W1)
    m1 = _mp_kernel(g1, src_r, dst_r, z128)
    g2 = _mid(m1, g1, dinvb, b1.reshape(1, H), W2)
    m2 = _mp_kernel(g2, src_r, dst_r, z128)
    g3 = _mid(m2, g2, dinvb, b2.reshape(1, H), W3)
    m3 = _mp_kernel(g3, src_r, dst_r, z128)
    out = _final(m3, g3, dinvb, b3.reshape(1, H), batch_p, Wl_p, bl_p)
    return out[:, :C]


# speedup vs baseline: 1.3461x; 1.0363x over previous
import functools
import jax
import jax.numpy as jnp
from jax import lax
from jax.experimental import pallas as pl
from jax.experimental.pallas import tpu as pltpu
from jax.experimental.pallas import tpu_sc as plsc

N = 10000
E = 320000
H = 128
G = 64
C = 10

NC = 2
NS = 16
NW = NC * NS
K = 128
CH = 80
NCH = NW * CH
EPAD = NCH * K
CHA = 48
CHB = 160 - CHA

NPAD = 10240
RPT = NPAD // NS
HRPT = RPT // 2
NB = 8
R = NPAD // NB

_mesh = plsc.VectorSubcoreMesh(core_axis_name="c", subcore_axis_name="s")


@functools.partial(
    pl.kernel,
    out_type=jax.ShapeDtypeStruct((NC, NPAD, H), jnp.float32),
    mesh=_mesh,
    scratch_types=[
        pltpu.VMEM((CH, K), jnp.int32),
        pltpu.VMEM((K, H), jnp.float32),
        pltpu.VMEM_SHARED((NPAD, H), jnp.float32),
    ],
)
def _deg_kernel(dst_hbm, ones_hbm, z_hbm, deg_out, dst_v, ones_v, acc):
    c = lax.axis_index("c")
    s = lax.axis_index("s")
    w = c * NS + s
    pltpu.sync_copy(dst_hbm.at[pl.ds(w * CH, CH)], dst_v)
    pltpu.sync_copy(ones_hbm, ones_v)
    pltpu.sync_copy(z_hbm, acc.at[pl.ds(s * RPT, RPT)])
    plsc.subcore_barrier()

    def body(j, carry):
        pltpu.sync_copy(ones_v, acc.at[dst_v.at[j]], add=True)
        return carry

    lax.fori_loop(0, CH, body, 0)
    plsc.subcore_barrier()
    pltpu.sync_copy(acc.at[pl.ds(s * RPT, RPT)], deg_out.at[c, pl.ds(s * RPT, RPT)])


W = 16


@functools.partial(
    pl.kernel,
    out_type=jax.ShapeDtypeStruct((NC, NPAD, H), jnp.float32),
    mesh=_mesh,
    scratch_types=[
        pltpu.VMEM((2, W, K), jnp.int32),
        pltpu.VMEM((2, W, K), jnp.int32),
        pltpu.VMEM((2, K, H), jnp.float32),
        pltpu.SemaphoreType.DMA,
        pltpu.SemaphoreType.DMA,
        pltpu.VMEM_SHARED((NPAD, H), jnp.float32),
    ],
)
def _mp_kernel(g_hbm, src_hbm, dst_hbm, z_hbm, m_out,
               src_w, dst_w, rows, sem0, sem1, acc):
    c = lax.axis_index("c")
    s = lax.axis_index("s")
    sems = (sem0, sem1)
    ch = jnp.where(c == 0, CHA, CHB)
    nwin = ch // W
    base = jnp.where(c == 0, s * CHA, NS * CHA + s * CHB)
    pltpu.sync_copy(src_hbm.at[pl.ds(base, W)], src_w.at[0])
    pltpu.sync_copy(dst_hbm.at[pl.ds(base, W)], dst_w.at[0])
    pltpu.sync_copy(z_hbm, acc.at[pl.ds(s * RPT, RPT)])
    plsc.subcore_barrier()

    @pl.when(ch > 0)
    def _():
        pltpu.async_copy(g_hbm.at[src_w.at[0, 0]], rows.at[0], sems[0])

    def win(v, carry):
        v2 = lax.rem(v, 2)
        v2n = lax.rem(v + 1, 2)

        @pl.when(v + 1 < nwin)
        def _():
            pltpu.sync_copy(src_hbm.at[pl.ds(base + (v + 1) * W, W)],
                            src_w.at[v2n])
            pltpu.sync_copy(dst_hbm.at[pl.ds(base + (v + 1) * W, W)],
                            dst_w.at[v2n])

        for t in range(W):
            j = v * W + t
            b, bn = t % 2, (t + 1) % 2
            pltpu.make_async_copy(
                g_hbm.at[src_w.at[v2, t]], rows.at[b], sems[b]).wait()
            nxt = (src_w.at[v2, t + 1] if t + 1 < W
                   else src_w.at[v2n, 0])

            @pl.when(j + 1 < ch)
            def _():
                pltpu.async_copy(g_hbm.at[nxt], rows.at[bn], sems[bn])

            pltpu.sync_copy(rows.at[b], acc.at[dst_w.at[v2, t]], add=True)
        return carry

    lax.fori_loop(0, nwin, win, 0)
    plsc.subcore_barrier()
    pltpu.sync_copy(acc.at[pl.ds(s * RPT, RPT)], m_out.at[c, pl.ds(s * RPT, RPT)])


def _pre_body(deg_ref, x_ref, w_ref, g_ref, dinv_ref):
    deg = deg_ref[0] + deg_ref[1] + 1.0
    dinvb = lax.rsqrt(deg)
    hw = jnp.dot(x_ref[...], w_ref[...], preferred_element_type=jnp.float32)
    g_ref[...] = dinvb * hw
    dinv_ref[...] = dinvb


_pre = pl.pallas_call(
    _pre_body,
    grid=(NB,),
    in_specs=[
        pl.BlockSpec((2, R, H), lambda i: (0, i, 0)),
        pl.BlockSpec((R, H), lambda i: (i, 0)),
        pl.BlockSpec((H, H), lambda i: (0, 0)),
    ],
    out_specs=[
        pl.BlockSpec((R, H), lambda i: (i, 0)),
        pl.BlockSpec((R, H), lambda i: (i, 0)),
    ],
    out_shape=[
        jax.ShapeDtypeStruct((NPAD, H), jnp.float32),
        jax.ShapeDtypeStruct((NPAD, H), jnp.float32),
    ],
)


def _mid_body(m_ref, g_ref, dinv_ref, b_ref, w_ref, o_ref):
    t = (m_ref[0] + m_ref[1] + g_ref[...]) * dinv_ref[...] + b_ref[...]
    h = jnp.maximum(t, 0.0)
    o_ref[...] = dinv_ref[...] * jnp.dot(
        h, w_ref[...], preferred_element_type=jnp.float32)


_mid = pl.pallas_call(
    _mid_body,
    grid=(NB,),
    in_specs=[
        pl.BlockSpec((2, R, H), lambda i: (0, i, 0)),
        pl.BlockSpec((R, H), lambda i: (i, 0)),
        pl.BlockSpec((R, H), lambda i: (i, 0)),
        pl.BlockSpec((1, H), lambda i: (0, 0)),
        pl.BlockSpec((H, H), lambda i: (0, 0)),
    ],
    out_specs=pl.BlockSpec((R, H), lambda i: (i, 0)),
    out_shape=jax.ShapeDtypeStruct((NPAD, H), jnp.float32),
)


def _final_body(m_ref, g_ref, dinv_ref, b_ref, bt_ref, wl_ref, bl_ref,
                o_ref, acc, cnt):
    i = pl.program_id(0)

    @pl.when(i == 0)
    def _():
        acc[...] = jnp.zeros_like(acc)
        cnt[...] = jnp.zeros_like(cnt)

    h = (m_ref[0] + m_ref[1] + g_ref[...]) * dinv_ref[...] + b_ref[...]
    bt = bt_ref[0]
    P = (lax.broadcasted_iota(jnp.int32, (G, R), 0) == bt
         ).astype(jnp.float32)
    acc[...] += jnp.dot(P, h, preferred_element_type=jnp.float32)
    cnt[...] += jnp.dot(P, jnp.ones((R, H), jnp.float32),
                        preferred_element_type=jnp.float32)

    @pl.when(i == NB - 1)
    def _():
        pooled = acc[...] / jnp.maximum(cnt[...], 1.0)
        o_ref[...] = jnp.dot(pooled, wl_ref[...],
                             preferred_element_type=jnp.float32) + bl_ref[...]


_final = pl.pallas_call(
    _final_body,
    grid=(NB,),
    in_specs=[
        pl.BlockSpec((2, R, H), lambda i: (0, i, 0)),
        pl.BlockSpec((R, H), lambda i: (i, 0)),
        pl.BlockSpec((R, H), lambda i: (i, 0)),
        pl.BlockSpec((1, H), lambda i: (0, 0)),
        pl.BlockSpec((1, 1, R), lambda i: (i, 0, 0)),
        pl.BlockSpec((H, H), lambda i: (0, 0)),
        pl.BlockSpec((1, H), lambda i: (0, 0)),
    ],
    out_specs=pl.BlockSpec((G, H), lambda i: (0, 0)),
    out_shape=jax.ShapeDtypeStruct((G, H), jnp.float32),
    scratch_shapes=[
        pltpu.VMEM((G, H), jnp.float32),
        pltpu.VMEM((G, H), jnp.float32),
    ],
)


def kernel(x, edge_index, batch, W1, b1, W2, b2, W3, b3, Wl, bl):
    src = edge_index[0]
    dst = edge_index[1]
    pad_e = EPAD - E
    src_r = jnp.concatenate(
        [src, jnp.zeros((pad_e,), jnp.int32)]).reshape(NCH, K)
    dst_r = jnp.concatenate(
        [dst, jnp.full((pad_e,), N, jnp.int32)]).reshape(NCH, K)

    x_pad = jnp.pad(x, ((0, NPAD - N), (0, 0)))
    batch_p = jnp.pad(batch, (0, NPAD - N),
                      constant_values=G).reshape(NB, 1, R)
    Wl_p = jnp.pad(Wl, ((0, 0), (0, H - C)))
    bl_p = jnp.pad(bl, (0, H - C)).reshape(1, H)

    ones128 = jnp.ones((K, H), jnp.float32)
    z128 = jnp.zeros((RPT, H), jnp.float32)

    deg_par = _deg_kernel(dst_r, ones128, z128)
    g1, dinvb = _pre(deg_par, x_pad, W1)
    m1 = _mp_kernel(g1, src_r, dst_r, z128)
    g2 = _mid(m1, g1, dinvb, b1.reshape(1, H), W2)
    m2 = _mp_kernel(g2, src_r, dst_r, z128)
    g3 = _mid(m2, g2, dinvb, b2.reshape(1, H), W3)
    m3 = _mp_kernel(g3, src_r, dst_r, z128)
    out = _final(m3, g3, dinvb, b3.reshape(1, H), batch_p, Wl_p, bl_p)
    return out[:, :C]


# speedup vs baseline: 1.3764x; 1.0225x over previous
import functools
import jax
import jax.numpy as jnp
from jax import lax
from jax.experimental import pallas as pl
from jax.experimental.pallas import tpu as pltpu
from jax.experimental.pallas import tpu_sc as plsc

N = 10000
E = 320000
H = 128
G = 64
C = 10

NC = 2
NS = 16
NW = NC * NS
K = 128
CH = 80
NCH = NW * CH
EPAD = NCH * K
CHA = 64
CHB = 160 - CHA

NPAD = 10240
RPT = NPAD // NS
HRPT = RPT // 2
NB = 8
R = NPAD // NB

_mesh = plsc.VectorSubcoreMesh(core_axis_name="c", subcore_axis_name="s")


@functools.partial(
    pl.kernel,
    out_type=jax.ShapeDtypeStruct((NC, NPAD, H), jnp.float32),
    mesh=_mesh,
    scratch_types=[
        pltpu.VMEM((CH, K), jnp.int32),
        pltpu.VMEM((K, H), jnp.float32),
        pltpu.VMEM_SHARED((NPAD, H), jnp.float32),
    ],
)
def _deg_kernel(dst_hbm, ones_hbm, z_hbm, deg_out, dst_v, ones_v, acc):
    c = lax.axis_index("c")
    s = lax.axis_index("s")
    w = c * NS + s
    pltpu.sync_copy(dst_hbm.at[pl.ds(w * CH, CH)], dst_v)
    pltpu.sync_copy(ones_hbm, ones_v)
    pltpu.sync_copy(z_hbm, acc.at[pl.ds(s * RPT, RPT)])
    plsc.subcore_barrier()

    def body(j, carry):
        pltpu.sync_copy(ones_v, acc.at[dst_v.at[j]], add=True)
        return carry

    lax.fori_loop(0, CH, body, 0)
    plsc.subcore_barrier()
    pltpu.sync_copy(acc.at[pl.ds(s * RPT, RPT)], deg_out.at[c, pl.ds(s * RPT, RPT)])


W = 16


@functools.partial(
    pl.kernel,
    out_type=jax.ShapeDtypeStruct((NC, NPAD, H), jnp.float32),
    mesh=_mesh,
    scratch_types=[
        pltpu.VMEM((2, W, K), jnp.int32),
        pltpu.VMEM((2, W, K), jnp.int32),
        pltpu.VMEM((2, K, H), jnp.float32),
        pltpu.SemaphoreType.DMA,
        pltpu.SemaphoreType.DMA,
        pltpu.VMEM_SHARED((NPAD, H), jnp.float32),
    ],
)
def _mp_kernel(g_hbm, src_hbm, dst_hbm, z_hbm, m_out,
               src_w, dst_w, rows, sem0, sem1, acc):
    c = lax.axis_index("c")
    s = lax.axis_index("s")
    sems = (sem0, sem1)
    ch = jnp.where(c == 0, CHA, CHB)
    nwin = ch // W
    base = jnp.where(c == 0, s * CHA, NS * CHA + s * CHB)
    pltpu.sync_copy(src_hbm.at[pl.ds(base, W)], src_w.at[0])
    pltpu.sync_copy(dst_hbm.at[pl.ds(base, W)], dst_w.at[0])
    pltpu.sync_copy(z_hbm, acc.at[pl.ds(s * RPT, RPT)])
    plsc.subcore_barrier()

    @pl.when(ch > 0)
    def _():
        pltpu.async_copy(g_hbm.at[src_w.at[0, 0]], rows.at[0], sems[0])

    def win(v, carry):
        v2 = lax.rem(v, 2)
        v2n = lax.rem(v + 1, 2)

        @pl.when(v + 1 < nwin)
        def _():
            pltpu.sync_copy(src_hbm.at[pl.ds(base + (v + 1) * W, W)],
                            src_w.at[v2n])
            pltpu.sync_copy(dst_hbm.at[pl.ds(base + (v + 1) * W, W)],
                            dst_w.at[v2n])

        for t in range(W):
            j = v * W + t
            b, bn = t % 2, (t + 1) % 2
            pltpu.make_async_copy(
                g_hbm.at[src_w.at[v2, t]], rows.at[b], sems[b]).wait()
            nxt = (src_w.at[v2, t + 1] if t + 1 < W
                   else src_w.at[v2n, 0])

            @pl.when(j + 1 < ch)
            def _():
                pltpu.async_copy(g_hbm.at[nxt], rows.at[bn], sems[bn])

            pltpu.sync_copy(rows.at[b], acc.at[dst_w.at[v2, t]], add=True)
        return carry

    lax.fori_loop(0, nwin, win, 0)
    plsc.subcore_barrier()
    pltpu.sync_copy(acc.at[pl.ds(s * RPT, RPT)], m_out.at[c, pl.ds(s * RPT, RPT)])


def _pre_body(deg_ref, x_ref, w_ref, g_ref, dinv_ref):
    deg = deg_ref[0] + deg_ref[1] + 1.0
    dinvb = lax.rsqrt(deg)
    hw = jnp.dot(x_ref[...], w_ref[...], preferred_element_type=jnp.float32)
    g_ref[...] = dinvb * hw
    dinv_ref[...] = dinvb


_pre = pl.pallas_call(
    _pre_body,
    grid=(NB,),
    in_specs=[
        pl.BlockSpec((2, R, H), lambda i: (0, i, 0)),
        pl.BlockSpec((R, H), lambda i: (i, 0)),
        pl.BlockSpec((H, H), lambda i: (0, 0)),
    ],
    out_specs=[
        pl.BlockSpec((R, H), lambda i: (i, 0)),
        pl.BlockSpec((R, H), lambda i: (i, 0)),
    ],
    out_shape=[
        jax.ShapeDtypeStruct((NPAD, H), jnp.float32),
        jax.ShapeDtypeStruct((NPAD, H), jnp.float32),
    ],
)


def _mid_body(m_ref, g_ref, dinv_ref, b_ref, w_ref, o_ref):
    t = (m_ref[0] + m_ref[1] + g_ref[...]) * dinv_ref[...] + b_ref[...]
    h = jnp.maximum(t, 0.0)
    o_ref[...] = dinv_ref[...] * jnp.dot(
        h, w_ref[...], preferred_element_type=jnp.float32)


_mid = pl.pallas_call(
    _mid_body,
    grid=(NB,),
    in_specs=[
        pl.BlockSpec((2, R, H), lambda i: (0, i, 0)),
        pl.BlockSpec((R, H), lambda i: (i, 0)),
        pl.BlockSpec((R, H), lambda i: (i, 0)),
        pl.BlockSpec((1, H), lambda i: (0, 0)),
        pl.BlockSpec((H, H), lambda i: (0, 0)),
    ],
    out_specs=pl.BlockSpec((R, H), lambda i: (i, 0)),
    out_shape=jax.ShapeDtypeStruct((NPAD, H), jnp.float32),
)


def _final_body(m_ref, g_ref, dinv_ref, b_ref, bt_ref, wl_ref, bl_ref,
                o_ref, acc, cnt):
    i = pl.program_id(0)

    @pl.when(i == 0)
    def _():
        acc[...] = jnp.zeros_like(acc)
        cnt[...] = jnp.zeros_like(cnt)

    h = (m_ref[0] + m_ref[1] + g_ref[...]) * dinv_ref[...] + b_ref[...]
    bt = bt_ref[0]
    P = (lax.broadcasted_iota(jnp.int32, (G, R), 0) == bt
         ).astype(jnp.float32)
    acc[...] += jnp.dot(P, h, preferred_element_type=jnp.float32)
    cnt[...] += jnp.dot(P, jnp.ones((R, H), jnp.float32),
                        preferred_element_type=jnp.float32)

    @pl.when(i == NB - 1)
    def _():
        pooled = acc[...] / jnp.maximum(cnt[...], 1.0)
        o_ref[...] = jnp.dot(pooled, wl_ref[...],
                             preferred_element_type=jnp.float32) + bl_ref[...]


_final = pl.pallas_call(
    _final_body,
    grid=(NB,),
    in_specs=[
        pl.BlockSpec((2, R, H), lambda i: (0, i, 0)),
        pl.BlockSpec((R, H), lambda i: (i, 0)),
        pl.BlockSpec((R, H), lambda i: (i, 0)),
        pl.BlockSpec((1, H), lambda i: (0, 0)),
        pl.BlockSpec((1, 1, R), lambda i: (i, 0, 0)),
        pl.BlockSpec((H, H), lambda i: (0, 0)),
        pl.BlockSpec((1, H), lambda i: (0, 0)),
    ],
    out_specs=pl.BlockSpec((G, H), lambda i: (0, 0)),
    out_shape=jax.ShapeDtypeStruct((G, H), jnp.float32),
    scratch_shapes=[
        pltpu.VMEM((G, H), jnp.float32),
        pltpu.VMEM((G, H), jnp.float32),
    ],
)


def kernel(x, edge_index, batch, W1, b1, W2, b2, W3, b3, Wl, bl):
    src = edge_index[0]
    dst = edge_index[1]
    pad_e = EPAD - E
    src_r = jnp.concatenate(
        [src, jnp.zeros((pad_e,), jnp.int32)]).reshape(NCH, K)
    dst_r = jnp.concatenate(
        [dst, jnp.full((pad_e,), N, jnp.int32)]).reshape(NCH, K)

    x_pad = jnp.pad(x, ((0, NPAD - N), (0, 0)))
    batch_p = jnp.pad(batch, (0, NPAD - N),
                      constant_values=G).reshape(NB, 1, R)
    Wl_p = jnp.pad(Wl, ((0, 0), (0, H - C)))
    bl_p = jnp.pad(bl, (0, H - C)).reshape(1, H)

    ones128 = jnp.ones((K, H), jnp.float32)
    z128 = jnp.zeros((RPT, H), jnp.float32)

    deg_par = _deg_kernel(dst_r, ones128, z128)
    g1, dinvb = _pre(deg_par, x_pad, W1)
    m1 = _mp_kernel(g1, src_r, dst_r, z128)
    g2 = _mid(m1, g1, dinvb, b1.reshape(1, H), W2)
    m2 = _mp_kernel(g2, src_r, dst_r, z128)
    g3 = _mid(m2, g2, dinvb, b2.reshape(1, H), W3)
    m3 = _mp_kernel(g3, src_r, dst_r, z128)
    out = _final(m3, g3, dinvb, b3.reshape(1, H), batch_p, Wl_p, bl_p)
    return out[:, :C]


# speedup vs baseline: 1.4053x; 1.0210x over previous
import functools
import jax
import jax.numpy as jnp
from jax import lax
from jax.experimental import pallas as pl
from jax.experimental.pallas import tpu as pltpu
from jax.experimental.pallas import tpu_sc as plsc

N = 10000
E = 320000
H = 128
G = 64
C = 10

NC = 2
NS = 16
NW = NC * NS
K = 128
CH = 80
NCH = NW * CH
EPAD = NCH * K
CHA = 80
CHB = 160 - CHA

NPAD = 10240
RPT = NPAD // NS
HRPT = RPT // 2
NB = 8
R = NPAD // NB

_mesh = plsc.VectorSubcoreMesh(core_axis_name="c", subcore_axis_name="s")


@functools.partial(
    pl.kernel,
    out_type=jax.ShapeDtypeStruct((NC, NPAD, H), jnp.float32),
    mesh=_mesh,
    scratch_types=[
        pltpu.VMEM((CH, K), jnp.int32),
        pltpu.VMEM((K, H), jnp.float32),
        pltpu.VMEM_SHARED((NPAD, H), jnp.float32),
    ],
)
def _deg_kernel(dst_hbm, ones_hbm, z_hbm, deg_out, dst_v, ones_v, acc):
    c = lax.axis_index("c")
    s = lax.axis_index("s")
    w = c * NS + s
    pltpu.sync_copy(dst_hbm.at[pl.ds(w * CH, CH)], dst_v)
    pltpu.sync_copy(ones_hbm, ones_v)
    pltpu.sync_copy(z_hbm, acc.at[pl.ds(s * RPT, RPT)])
    plsc.subcore_barrier()

    def body(j, carry):
        pltpu.sync_copy(ones_v, acc.at[dst_v.at[j]], add=True)
        return carry

    lax.fori_loop(0, CH, body, 0)
    plsc.subcore_barrier()
    pltpu.sync_copy(acc.at[pl.ds(s * RPT, RPT)], deg_out.at[c, pl.ds(s * RPT, RPT)])


W = 16


@functools.partial(
    pl.kernel,
    out_type=jax.ShapeDtypeStruct((NC, NPAD, H), jnp.float32),
    mesh=_mesh,
    scratch_types=[
        pltpu.VMEM((2, W, K), jnp.int32),
        pltpu.VMEM((2, W, K), jnp.int32),
        pltpu.VMEM((2, K, H), jnp.float32),
        pltpu.SemaphoreType.DMA,
        pltpu.SemaphoreType.DMA,
        pltpu.VMEM_SHARED((NPAD, H), jnp.float32),
    ],
)
def _mp_kernel(g_hbm, src_hbm, dst_hbm, z_hbm, m_out,
               src_w, dst_w, rows, sem0, sem1, acc):
    c = lax.axis_index("c")
    s = lax.axis_index("s")
    sems = (sem0, sem1)
    ch = jnp.where(c == 0, CHA, CHB)
    nwin = ch // W
    base = jnp.where(c == 0, s * CHA, NS * CHA + s * CHB)
    pltpu.sync_copy(src_hbm.at[pl.ds(base, W)], src_w.at[0])
    pltpu.sync_copy(dst_hbm.at[pl.ds(base, W)], dst_w.at[0])
    pltpu.sync_copy(z_hbm, acc.at[pl.ds(s * RPT, RPT)])
    plsc.subcore_barrier()

    @pl.when(ch > 0)
    def _():
        pltpu.async_copy(g_hbm.at[src_w.at[0, 0]], rows.at[0], sems[0])

    def win(v, carry):
        v2 = lax.rem(v, 2)
        v2n = lax.rem(v + 1, 2)

        @pl.when(v + 1 < nwin)
        def _():
            pltpu.sync_copy(src_hbm.at[pl.ds(base + (v + 1) * W, W)],
                            src_w.at[v2n])
            pltpu.sync_copy(dst_hbm.at[pl.ds(base + (v + 1) * W, W)],
                            dst_w.at[v2n])

        for t in range(W):
            j = v * W + t
            b, bn = t % 2, (t + 1) % 2
            pltpu.make_async_copy(
                g_hbm.at[src_w.at[v2, t]], rows.at[b], sems[b]).wait()
            nxt = (src_w.at[v2, t + 1] if t + 1 < W
                   else src_w.at[v2n, 0])

            @pl.when(j + 1 < ch)
            def _():
                pltpu.async_copy(g_hbm.at[nxt], rows.at[bn], sems[bn])

            pltpu.sync_copy(rows.at[b], acc.at[dst_w.at[v2, t]], add=True)
        return carry

    lax.fori_loop(0, nwin, win, 0)
    plsc.subcore_barrier()
    pltpu.sync_copy(acc.at[pl.ds(s * RPT, RPT)], m_out.at[c, pl.ds(s * RPT, RPT)])


def _pre_body(deg_ref, x_ref, w_ref, g_ref, dinv_ref):
    deg = deg_ref[0] + deg_ref[1] + 1.0
    dinvb = lax.rsqrt(deg)
    hw = jnp.dot(x_ref[...], w_ref[...], preferred_element_type=jnp.float32)
    g_ref[...] = dinvb * hw
    dinv_ref[...] = dinvb


_pre = pl.pallas_call(
    _pre_body,
    grid=(NB,),
    in_specs=[
        pl.BlockSpec((2, R, H), lambda i: (0, i, 0)),
        pl.BlockSpec((R, H), lambda i: (i, 0)),
        pl.BlockSpec((H, H), lambda i: (0, 0)),
    ],
    out_specs=[
        pl.BlockSpec((R, H), lambda i: (i, 0)),
        pl.BlockSpec((R, H), lambda i: (i, 0)),
    ],
    out_shape=[
        jax.ShapeDtypeStruct((NPAD, H), jnp.float32),
        jax.ShapeDtypeStruct((NPAD, H), jnp.float32),
    ],
)


def _mid_body(m_ref, g_ref, dinv_ref, b_ref, w_ref, o_ref):
    t = (m_ref[0] + m_ref[1] + g_ref[...]) * dinv_ref[...] + b_ref[...]
    h = jnp.maximum(t, 0.0)
    o_ref[...] = dinv_ref[...] * jnp.dot(
        h, w_ref[...], preferred_element_type=jnp.float32)


_mid = pl.pallas_call(
    _mid_body,
    grid=(NB,),
    in_specs=[
        pl.BlockSpec((2, R, H), lambda i: (0, i, 0)),
        pl.BlockSpec((R, H), lambda i: (i, 0)),
        pl.BlockSpec((R, H), lambda i: (i, 0)),
        pl.BlockSpec((1, H), lambda i: (0, 0)),
        pl.BlockSpec((H, H), lambda i: (0, 0)),
    ],
    out_specs=pl.BlockSpec((R, H), lambda i: (i, 0)),
    out_shape=jax.ShapeDtypeStruct((NPAD, H), jnp.float32),
)


def _final_body(m_ref, g_ref, dinv_ref, b_ref, bt_ref, wl_ref, bl_ref,
                o_ref, acc, cnt):
    i = pl.program_id(0)

    @pl.when(i == 0)
    def _():
        acc[...] = jnp.zeros_like(acc)
        cnt[...] = jnp.zeros_like(cnt)

    h = (m_ref[0] + m_ref[1] + g_ref[...]) * dinv_ref[...] + b_ref[...]
    bt = bt_ref[0]
    P = (lax.broadcasted_iota(jnp.int32, (G, R), 0) == bt
         ).astype(jnp.float32)
    acc[...] += jnp.dot(P, h, preferred_element_type=jnp.float32)
    cnt[...] += jnp.dot(P, jnp.ones((R, H), jnp.float32),
                        preferred_element_type=jnp.float32)

    @pl.when(i == NB - 1)
    def _():
        pooled = acc[...] / jnp.maximum(cnt[...], 1.0)
        o_ref[...] = jnp.dot(pooled, wl_ref[...],
                             preferred_element_type=jnp.float32) + bl_ref[...]


_final = pl.pallas_call(
    _final_body,
    grid=(NB,),
    in_specs=[
        pl.BlockSpec((2, R, H), lambda i: (0, i, 0)),
        pl.BlockSpec((R, H), lambda i: (i, 0)),
        pl.BlockSpec((R, H), lambda i: (i, 0)),
        pl.BlockSpec((1, H), lambda i: (0, 0)),
        pl.BlockSpec((1, 1, R), lambda i: (i, 0, 0)),
        pl.BlockSpec((H, H), lambda i: (0, 0)),
        pl.BlockSpec((1, H), lambda i: (0, 0)),
    ],
    out_specs=pl.BlockSpec((G, H), lambda i: (0, 0)),
    out_shape=jax.ShapeDtypeStruct((G, H), jnp.float32),
    scratch_shapes=[
        pltpu.VMEM((G, H), jnp.float32),
        pltpu.VMEM((G, H), jnp.float32),
    ],
)


def kernel(x, edge_index, batch, W1, b1, W2, b2, W3, b3, Wl, bl):
    src = edge_index[0]
    dst = edge_index[1]
    pad_e = EPAD - E
    src_r = jnp.concatenate(
        [src, jnp.zeros((pad_e,), jnp.int32)]).reshape(NCH, K)
    dst_r = jnp.concatenate(
        [dst, jnp.full((pad_e,), N, jnp.int32)]).reshape(NCH, K)

    x_pad = jnp.pad(x, ((0, NPAD - N), (0, 0)))
    batch_p = jnp.pad(batch, (0, NPAD - N),
                      constant_values=G).reshape(NB, 1, R)
    Wl_p = jnp.pad(Wl, ((0, 0), (0, H - C)))
    bl_p = jnp.pad(bl, (0, H - C)).reshape(1, H)

    ones128 = jnp.ones((K, H), jnp.float32)
    z128 = jnp.zeros((RPT, H), jnp.float32)

    deg_par = _deg_kernel(dst_r, ones128, z128)
    g1, dinvb = _pre(deg_par, x_pad, W1)
    m1 = _mp_kernel(g1, src_r, dst_r, z128)
    g2 = _mid(m1, g1, dinvb, b1.reshape(1, H), W2)
    m2 = _mp_kernel(g2, src_r, dst_r, z128)
    g3 = _mid(m2, g2, dinvb, b2.reshape(1, H), W3)
    m3 = _mp_kernel(g3, src_r, dst_r, z128)
    out = _final(m3, g3, dinvb, b3.reshape(1, H), batch_p, Wl_p, bl_p)
    return out[:, :C]


# speedup vs baseline: 1.4603x; 1.0391x over previous
import functools
import jax
import jax.numpy as jnp
from jax import lax
from jax.experimental import pallas as pl
from jax.experimental.pallas import tpu as pltpu
from jax.experimental.pallas import tpu_sc as plsc

N = 10000
E = 320000
H = 128
G = 64
C = 10

NC = 2
NS = 16
NW = NC * NS
K = 128
CH = 80
NCH = NW * CH
EPAD = NCH * K
CHA = 96
CHB = 160 - CHA

NPAD = 10240
RPT = NPAD // NS
HRPT = RPT // 2
NB = 8
R = NPAD // NB

_mesh = plsc.VectorSubcoreMesh(core_axis_name="c", subcore_axis_name="s")


@functools.partial(
    pl.kernel,
    out_type=jax.ShapeDtypeStruct((NC, NPAD, H), jnp.float32),
    mesh=_mesh,
    scratch_types=[
        pltpu.VMEM((CH, K), jnp.int32),
        pltpu.VMEM((K, H), jnp.float32),
        pltpu.VMEM_SHARED((NPAD, H), jnp.float32),
    ],
)
def _deg_kernel(dst_hbm, ones_hbm, z_hbm, deg_out, dst_v, ones_v, acc):
    c = lax.axis_index("c")
    s = lax.axis_index("s")
    w = c * NS + s
    pltpu.sync_copy(dst_hbm.at[pl.ds(w * CH, CH)], dst_v)
    pltpu.sync_copy(ones_hbm, ones_v)
    pltpu.sync_copy(z_hbm, acc.at[pl.ds(s * RPT, RPT)])
    plsc.subcore_barrier()

    def body(j, carry):
        pltpu.sync_copy(ones_v, acc.at[dst_v.at[j]], add=True)
        return carry

    lax.fori_loop(0, CH, body, 0)
    plsc.subcore_barrier()
    pltpu.sync_copy(acc.at[pl.ds(s * RPT, RPT)], deg_out.at[c, pl.ds(s * RPT, RPT)])


W = 16


@functools.partial(
    pl.kernel,
    out_type=jax.ShapeDtypeStruct((NC, NPAD, H), jnp.float32),
    mesh=_mesh,
    scratch_types=[
        pltpu.VMEM((2, W, K), jnp.int32),
        pltpu.VMEM((2, W, K), jnp.int32),
        pltpu.VMEM((2, K, H), jnp.float32),
        pltpu.SemaphoreType.DMA,
        pltpu.SemaphoreType.DMA,
        pltpu.VMEM_SHARED((NPAD, H), jnp.float32),
    ],
)
def _mp_kernel(g_hbm, src_hbm, dst_hbm, z_hbm, m_out,
               src_w, dst_w, rows, sem0, sem1, acc):
    c = lax.axis_index("c")
    s = lax.axis_index("s")
    sems = (sem0, sem1)
    ch = jnp.where(c == 0, CHA, CHB)
    nwin = ch // W
    base = jnp.where(c == 0, s * CHA, NS * CHA + s * CHB)
    pltpu.sync_copy(src_hbm.at[pl.ds(base, W)], src_w.at[0])
    pltpu.sync_copy(dst_hbm.at[pl.ds(base, W)], dst_w.at[0])
    pltpu.sync_copy(z_hbm, acc.at[pl.ds(s * RPT, RPT)])
    plsc.subcore_barrier()

    @pl.when(ch > 0)
    def _():
        pltpu.async_copy(g_hbm.at[src_w.at[0, 0]], rows.at[0], sems[0])

    def win(v, carry):
        v2 = lax.rem(v, 2)
        v2n = lax.rem(v + 1, 2)

        @pl.when(v + 1 < nwin)
        def _():
            pltpu.sync_copy(src_hbm.at[pl.ds(base + (v + 1) * W, W)],
                            src_w.at[v2n])
            pltpu.sync_copy(dst_hbm.at[pl.ds(base + (v + 1) * W, W)],
                            dst_w.at[v2n])

        for t in range(W):
            j = v * W + t
            b, bn = t % 2, (t + 1) % 2
            pltpu.make_async_copy(
                g_hbm.at[src_w.at[v2, t]], rows.at[b], sems[b]).wait()
            nxt = (src_w.at[v2, t + 1] if t + 1 < W
                   else src_w.at[v2n, 0])

            @pl.when(j + 1 < ch)
            def _():
                pltpu.async_copy(g_hbm.at[nxt], rows.at[bn], sems[bn])

            pltpu.sync_copy(rows.at[b], acc.at[dst_w.at[v2, t]], add=True)
        return carry

    lax.fori_loop(0, nwin, win, 0)
    plsc.subcore_barrier()
    pltpu.sync_copy(acc.at[pl.ds(s * RPT, RPT)], m_out.at[c, pl.ds(s * RPT, RPT)])


def _pre_body(deg_ref, x_ref, w_ref, g_ref, dinv_ref):
    deg = deg_ref[0] + deg_ref[1] + 1.0
    dinvb = lax.rsqrt(deg)
    hw = jnp.dot(x_ref[...], w_ref[...], preferred_element_type=jnp.float32)
    g_ref[...] = dinvb * hw
    dinv_ref[...] = dinvb


_pre = pl.pallas_call(
    _pre_body,
    grid=(NB,),
    in_specs=[
        pl.BlockSpec((2, R, H), lambda i: (0, i, 0)),
        pl.BlockSpec((R, H), lambda i: (i, 0)),
        pl.BlockSpec((H, H), lambda i: (0, 0)),
    ],
    out_specs=[
        pl.BlockSpec((R, H), lambda i: (i, 0)),
        pl.BlockSpec((R, H), lambda i: (i, 0)),
    ],
    out_shape=[
        jax.ShapeDtypeStruct((NPAD, H), jnp.float32),
        jax.ShapeDtypeStruct((NPAD, H), jnp.float32),
    ],
)


def _mid_body(m_ref, g_ref, dinv_ref, b_ref, w_ref, o_ref):
    t = (m_ref[0] + m_ref[1] + g_ref[...]) * dinv_ref[...] + b_ref[...]
    h = jnp.maximum(t, 0.0)
    o_ref[...] = dinv_ref[...] * jnp.dot(
        h, w_ref[...], preferred_element_type=jnp.float32)


_mid = pl.pallas_call(
    _mid_body,
    grid=(NB,),
    in_specs=[
        pl.BlockSpec((2, R, H), lambda i: (0, i, 0)),
        pl.BlockSpec((R, H), lambda i: (i, 0)),
        pl.BlockSpec((R, H), lambda i: (i, 0)),
        pl.BlockSpec((1, H), lambda i: (0, 0)),
        pl.BlockSpec((H, H), lambda i: (0, 0)),
    ],
    out_specs=pl.BlockSpec((R, H), lambda i: (i, 0)),
    out_shape=jax.ShapeDtypeStruct((NPAD, H), jnp.float32),
)


def _final_body(m_ref, g_ref, dinv_ref, b_ref, bt_ref, wl_ref, bl_ref,
                o_ref, acc, cnt):
    i = pl.program_id(0)

    @pl.when(i == 0)
    def _():
        acc[...] = jnp.zeros_like(acc)
        cnt[...] = jnp.zeros_like(cnt)

    h = (m_ref[0] + m_ref[1] + g_ref[...]) * dinv_ref[...] + b_ref[...]
    bt = bt_ref[0]
    P = (lax.broadcasted_iota(jnp.int32, (G, R), 0) == bt
         ).astype(jnp.float32)
    acc[...] += jnp.dot(P, h, preferred_element_type=jnp.float32)
    cnt[...] += jnp.dot(P, jnp.ones((R, H), jnp.float32),
                        preferred_element_type=jnp.float32)

    @pl.when(i == NB - 1)
    def _():
        pooled = acc[...] / jnp.maximum(cnt[...], 1.0)
        o_ref[...] = jnp.dot(pooled, wl_ref[...],
                             preferred_element_type=jnp.float32) + bl_ref[...]


_final = pl.pallas_call(
    _final_body,
    grid=(NB,),
    in_specs=[
        pl.BlockSpec((2, R, H), lambda i: (0, i, 0)),
        pl.BlockSpec((R, H), lambda i: (i, 0)),
        pl.BlockSpec((R, H), lambda i: (i, 0)),
        pl.BlockSpec((1, H), lambda i: (0, 0)),
        pl.BlockSpec((1, 1, R), lambda i: (i, 0, 0)),
        pl.BlockSpec((H, H), lambda i: (0, 0)),
        pl.BlockSpec((1, H), lambda i: (0, 0)),
    ],
    out_specs=pl.BlockSpec((G, H), lambda i: (0, 0)),
    out_shape=jax.ShapeDtypeStruct((G, H), jnp.float32),
    scratch_shapes=[
        pltpu.VMEM((G, H), jnp.float32),
        pltpu.VMEM((G, H), jnp.float32),
    ],
)


def kernel(x, edge_index, batch, W1, b1, W2, b2, W3, b3, Wl, bl):
    src = edge_index[0]
    dst = edge_index[1]
    pad_e = EPAD - E
    src_r = jnp.concatenate(
        [src, jnp.zeros((pad_e,), jnp.int32)]).reshape(NCH, K)
    dst_r = jnp.concatenate(
        [dst, jnp.full((pad_e,), N, jnp.int32)]).reshape(NCH, K)

    x_pad = jnp.pad(x, ((0, NPAD - N), (0, 0)))
    batch_p = jnp.pad(batch, (0, NPAD - N),
                      constant_values=G).reshape(NB, 1, R)
    Wl_p = jnp.pad(Wl, ((0, 0), (0, H - C)))
    bl_p = jnp.pad(bl, (0, H - C)).reshape(1, H)

    ones128 = jnp.ones((K, H), jnp.float32)
    z128 = jnp.zeros((RPT, H), jnp.float32)

    deg_par = _deg_kernel(dst_r, ones128, z128)
    g1, dinvb = _pre(deg_par, x_pad, W1)
    m1 = _mp_kernel(g1, src_r, dst_r, z128)
    g2 = _mid(m1, g1, dinvb, b1.reshape(1, H), W2)
    m2 = _mp_kernel(g2, src_r, dst_r, z128)
    g3 = _mid(m2, g2, dinvb, b2.reshape(1, H), W3)
    m3 = _mp_kernel(g3, src_r, dst_r, z128)
    out = _final(m3, g3, dinvb, b3.reshape(1, H), batch_p, Wl_p, bl_p)
    return out[:, :C]


# speedup vs baseline: 1.5176x; 1.0393x over previous
import functools
import jax
import jax.numpy as jnp
from jax import lax
from jax.experimental import pallas as pl
from jax.experimental.pallas import tpu as pltpu
from jax.experimental.pallas import tpu_sc as plsc

N = 10000
E = 320000
H = 128
G = 64
C = 10

NC = 2
NS = 16
NW = NC * NS
K = 128
CH = 80
NCH = NW * CH
EPAD = NCH * K
CHA = 112
CHB = 160 - CHA

NPAD = 10240
RPT = NPAD // NS
HRPT = RPT // 2
NB = 8
R = NPAD // NB

_mesh = plsc.VectorSubcoreMesh(core_axis_name="c", subcore_axis_name="s")


@functools.partial(
    pl.kernel,
    out_type=jax.ShapeDtypeStruct((NC, NPAD, H), jnp.float32),
    mesh=_mesh,
    scratch_types=[
        pltpu.VMEM((CH, K), jnp.int32),
        pltpu.VMEM((K, H), jnp.float32),
        pltpu.VMEM_SHARED((NPAD, H), jnp.float32),
    ],
)
def _deg_kernel(dst_hbm, ones_hbm, z_hbm, deg_out, dst_v, ones_v, acc):
    c = lax.axis_index("c")
    s = lax.axis_index("s")
    w = c * NS + s
    pltpu.sync_copy(dst_hbm.at[pl.ds(w * CH, CH)], dst_v)
    pltpu.sync_copy(ones_hbm, ones_v)
    pltpu.sync_copy(z_hbm, acc.at[pl.ds(s * RPT, RPT)])
    plsc.subcore_barrier()

    def body(j, carry):
        pltpu.sync_copy(ones_v, acc.at[dst_v.at[j]], add=True)
        return carry

    lax.fori_loop(0, CH, body, 0)
    plsc.subcore_barrier()
    pltpu.sync_copy(acc.at[pl.ds(s * RPT, RPT)], deg_out.at[c, pl.ds(s * RPT, RPT)])


W = 16


@functools.partial(
    pl.kernel,
    out_type=jax.ShapeDtypeStruct((NC, NPAD, H), jnp.float32),
    mesh=_mesh,
    scratch_types=[
        pltpu.VMEM((2, W, K), jnp.int32),
        pltpu.VMEM((2, W, K), jnp.int32),
        pltpu.VMEM((2, K, H), jnp.float32),
        pltpu.SemaphoreType.DMA,
        pltpu.SemaphoreType.DMA,
        pltpu.VMEM_SHARED((NPAD, H), jnp.float32),
    ],
)
def _mp_kernel(g_hbm, src_hbm, dst_hbm, z_hbm, m_out,
               src_w, dst_w, rows, sem0, sem1, acc):
    c = lax.axis_index("c")
    s = lax.axis_index("s")
    sems = (sem0, sem1)
    ch = jnp.where(c == 0, CHA, CHB)
    nwin = ch // W
    base = jnp.where(c == 0, s * CHA, NS * CHA + s * CHB)
    pltpu.sync_copy(src_hbm.at[pl.ds(base, W)], src_w.at[0])
    pltpu.sync_copy(dst_hbm.at[pl.ds(base, W)], dst_w.at[0])
    pltpu.sync_copy(z_hbm, acc.at[pl.ds(s * RPT, RPT)])
    plsc.subcore_barrier()

    @pl.when(ch > 0)
    def _():
        pltpu.async_copy(g_hbm.at[src_w.at[0, 0]], rows.at[0], sems[0])

    def win(v, carry):
        v2 = lax.rem(v, 2)
        v2n = lax.rem(v + 1, 2)

        @pl.when(v + 1 < nwin)
        def _():
            pltpu.sync_copy(src_hbm.at[pl.ds(base + (v + 1) * W, W)],
                            src_w.at[v2n])
            pltpu.sync_copy(dst_hbm.at[pl.ds(base + (v + 1) * W, W)],
                            dst_w.at[v2n])

        for t in range(W):
            j = v * W + t
            b, bn = t % 2, (t + 1) % 2
            pltpu.make_async_copy(
                g_hbm.at[src_w.at[v2, t]], rows.at[b], sems[b]).wait()
            nxt = (src_w.at[v2, t + 1] if t + 1 < W
                   else src_w.at[v2n, 0])

            @pl.when(j + 1 < ch)
            def _():
                pltpu.async_copy(g_hbm.at[nxt], rows.at[bn], sems[bn])

            pltpu.sync_copy(rows.at[b], acc.at[dst_w.at[v2, t]], add=True)
        return carry

    lax.fori_loop(0, nwin, win, 0)
    plsc.subcore_barrier()
    pltpu.sync_copy(acc.at[pl.ds(s * RPT, RPT)], m_out.at[c, pl.ds(s * RPT, RPT)])


def _pre_body(deg_ref, x_ref, w_ref, g_ref, dinv_ref):
    deg = deg_ref[0] + deg_ref[1] + 1.0
    dinvb = lax.rsqrt(deg)
    hw = jnp.dot(x_ref[...], w_ref[...], preferred_element_type=jnp.float32)
    g_ref[...] = dinvb * hw
    dinv_ref[...] = dinvb


_pre = pl.pallas_call(
    _pre_body,
    grid=(NB,),
    in_specs=[
        pl.BlockSpec((2, R, H), lambda i: (0, i, 0)),
        pl.BlockSpec((R, H), lambda i: (i, 0)),
        pl.BlockSpec((H, H), lambda i: (0, 0)),
    ],
    out_specs=[
        pl.BlockSpec((R, H), lambda i: (i, 0)),
        pl.BlockSpec((R, H), lambda i: (i, 0)),
    ],
    out_shape=[
        jax.ShapeDtypeStruct((NPAD, H), jnp.float32),
        jax.ShapeDtypeStruct((NPAD, H), jnp.float32),
    ],
)


def _mid_body(m_ref, g_ref, dinv_ref, b_ref, w_ref, o_ref):
    t = (m_ref[0] + m_ref[1] + g_ref[...]) * dinv_ref[...] + b_ref[...]
    h = jnp.maximum(t, 0.0)
    o_ref[...] = dinv_ref[...] * jnp.dot(
        h, w_ref[...], preferred_element_type=jnp.float32)


_mid = pl.pallas_call(
    _mid_body,
    grid=(NB,),
    in_specs=[
        pl.BlockSpec((2, R, H), lambda i: (0, i, 0)),
        pl.BlockSpec((R, H), lambda i: (i, 0)),
        pl.BlockSpec((R, H), lambda i: (i, 0)),
        pl.BlockSpec((1, H), lambda i: (0, 0)),
        pl.BlockSpec((H, H), lambda i: (0, 0)),
    ],
    out_specs=pl.BlockSpec((R, H), lambda i: (i, 0)),
    out_shape=jax.ShapeDtypeStruct((NPAD, H), jnp.float32),
)


def _final_body(m_ref, g_ref, dinv_ref, b_ref, bt_ref, wl_ref, bl_ref,
                o_ref, acc, cnt):
    i = pl.program_id(0)

    @pl.when(i == 0)
    def _():
        acc[...] = jnp.zeros_like(acc)
        cnt[...] = jnp.zeros_like(cnt)

    h = (m_ref[0] + m_ref[1] + g_ref[...]) * dinv_ref[...] + b_ref[...]
    bt = bt_ref[0]
    P = (lax.broadcasted_iota(jnp.int32, (G, R), 0) == bt
         ).astype(jnp.float32)
    acc[...] += jnp.dot(P, h, preferred_element_type=jnp.float32)
    cnt[...] += jnp.dot(P, jnp.ones((R, H), jnp.float32),
                        preferred_element_type=jnp.float32)

    @pl.when(i == NB - 1)
    def _():
        pooled = acc[...] / jnp.maximum(cnt[...], 1.0)
        o_ref[...] = jnp.dot(pooled, wl_ref[...],
                             preferred_element_type=jnp.float32) + bl_ref[...]


_final = pl.pallas_call(
    _final_body,
    grid=(NB,),
    in_specs=[
        pl.BlockSpec((2, R, H), lambda i: (0, i, 0)),
        pl.BlockSpec((R, H), lambda i: (i, 0)),
        pl.BlockSpec((R, H), lambda i: (i, 0)),
        pl.BlockSpec((1, H), lambda i: (0, 0)),
        pl.BlockSpec((1, 1, R), lambda i: (i, 0, 0)),
        pl.BlockSpec((H, H), lambda i: (0, 0)),
        pl.BlockSpec((1, H), lambda i: (0, 0)),
    ],
    out_specs=pl.BlockSpec((G, H), lambda i: (0, 0)),
    out_shape=jax.ShapeDtypeStruct((G, H), jnp.float32),
    scratch_shapes=[
        pltpu.VMEM((G, H), jnp.float32),
        pltpu.VMEM((G, H), jnp.float32),
    ],
)


def kernel(x, edge_index, batch, W1, b1, W2, b2, W3, b3, Wl, bl):
    src = edge_index[0]
    dst = edge_index[1]
    pad_e = EPAD - E
    src_r = jnp.concatenate(
        [src, jnp.zeros((pad_e,), jnp.int32)]).reshape(NCH, K)
    dst_r = jnp.concatenate(
        [dst, jnp.full((pad_e,), N, jnp.int32)]).reshape(NCH, K)

    x_pad = jnp.pad(x, ((0, NPAD - N), (0, 0)))
    batch_p = jnp.pad(batch, (0, NPAD - N),
                      constant_values=G).reshape(NB, 1, R)
    Wl_p = jnp.pad(Wl, ((0, 0), (0, H - C)))
    bl_p = jnp.pad(bl, (0, H - C)).reshape(1, H)

    ones128 = jnp.ones((K, H), jnp.float32)
    z128 = jnp.zeros((RPT, H), jnp.float32)

    deg_par = _deg_kernel(dst_r, ones128, z128)
    g1, dinvb = _pre(deg_par, x_pad, W1)
    m1 = _mp_kernel(g1, src_r, dst_r, z128)
    g2 = _mid(m1, g1, dinvb, b1.reshape(1, H), W2)
    m2 = _mp_kernel(g2, src_r, dst_r, z128)
    g3 = _mid(m2, g2, dinvb, b2.reshape(1, H), W3)
    m3 = _mp_kernel(g3, src_r, dst_r, z128)
    out = _final(m3, g3, dinvb, b3.reshape(1, H), batch_p, Wl_p, bl_p)
    return out[:, :C]


# speedup vs baseline: 1.5670x; 1.0325x over previous
import functools
import jax
import jax.numpy as jnp
from jax import lax
from jax.experimental import pallas as pl
from jax.experimental.pallas import tpu as pltpu
from jax.experimental.pallas import tpu_sc as plsc

N = 10000
E = 320000
H = 128
G = 64
C = 10

NC = 2
NS = 16
NW = NC * NS
K = 128
CH = 80
NCH = NW * CH
EPAD = NCH * K
CHA = 128
CHB = 160 - CHA

NPAD = 10240
RPT = NPAD // NS
HRPT = RPT // 2
NB = 8
R = NPAD // NB

_mesh = plsc.VectorSubcoreMesh(core_axis_name="c", subcore_axis_name="s")


@functools.partial(
    pl.kernel,
    out_type=jax.ShapeDtypeStruct((NC, NPAD, H), jnp.float32),
    mesh=_mesh,
    scratch_types=[
        pltpu.VMEM((CH, K), jnp.int32),
        pltpu.VMEM((K, H), jnp.float32),
        pltpu.VMEM_SHARED((NPAD, H), jnp.float32),
    ],
)
def _deg_kernel(dst_hbm, ones_hbm, z_hbm, deg_out, dst_v, ones_v, acc):
    c = lax.axis_index("c")
    s = lax.axis_index("s")
    w = c * NS + s
    pltpu.sync_copy(dst_hbm.at[pl.ds(w * CH, CH)], dst_v)
    pltpu.sync_copy(ones_hbm, ones_v)
    pltpu.sync_copy(z_hbm, acc.at[pl.ds(s * RPT, RPT)])
    plsc.subcore_barrier()

    def body(j, carry):
        pltpu.sync_copy(ones_v, acc.at[dst_v.at[j]], add=True)
        return carry

    lax.fori_loop(0, CH, body, 0)
    plsc.subcore_barrier()
    pltpu.sync_copy(acc.at[pl.ds(s * RPT, RPT)], deg_out.at[c, pl.ds(s * RPT, RPT)])


W = 16


@functools.partial(
    pl.kernel,
    out_type=jax.ShapeDtypeStruct((NC, NPAD, H), jnp.float32),
    mesh=_mesh,
    scratch_types=[
        pltpu.VMEM((2, W, K), jnp.int32),
        pltpu.VMEM((2, W, K), jnp.int32),
        pltpu.VMEM((2, K, H), jnp.float32),
        pltpu.SemaphoreType.DMA,
        pltpu.SemaphoreType.DMA,
        pltpu.VMEM_SHARED((NPAD, H), jnp.float32),
    ],
)
def _mp_kernel(g_hbm, src_hbm, dst_hbm, z_hbm, m_out,
               src_w, dst_w, rows, sem0, sem1, acc):
    c = lax.axis_index("c")
    s = lax.axis_index("s")
    sems = (sem0, sem1)
    ch = jnp.where(c == 0, CHA, CHB)
    nwin = ch // W
    base = jnp.where(c == 0, s * CHA, NS * CHA + s * CHB)
    pltpu.sync_copy(src_hbm.at[pl.ds(base, W)], src_w.at[0])
    pltpu.sync_copy(dst_hbm.at[pl.ds(base, W)], dst_w.at[0])
    pltpu.sync_copy(z_hbm, acc.at[pl.ds(s * RPT, RPT)])
    plsc.subcore_barrier()

    @pl.when(ch > 0)
    def _():
        pltpu.async_copy(g_hbm.at[src_w.at[0, 0]], rows.at[0], sems[0])

    def win(v, carry):
        v2 = lax.rem(v, 2)
        v2n = lax.rem(v + 1, 2)

        @pl.when(v + 1 < nwin)
        def _():
            pltpu.sync_copy(src_hbm.at[pl.ds(base + (v + 1) * W, W)],
                            src_w.at[v2n])
            pltpu.sync_copy(dst_hbm.at[pl.ds(base + (v + 1) * W, W)],
                            dst_w.at[v2n])

        for t in range(W):
            j = v * W + t
            b, bn = t % 2, (t + 1) % 2
            pltpu.make_async_copy(
                g_hbm.at[src_w.at[v2, t]], rows.at[b], sems[b]).wait()
            nxt = (src_w.at[v2, t + 1] if t + 1 < W
                   else src_w.at[v2n, 0])

            @pl.when(j + 1 < ch)
            def _():
                pltpu.async_copy(g_hbm.at[nxt], rows.at[bn], sems[bn])

            pltpu.sync_copy(rows.at[b], acc.at[dst_w.at[v2, t]], add=True)
        return carry

    lax.fori_loop(0, nwin, win, 0)
    plsc.subcore_barrier()
    pltpu.sync_copy(acc.at[pl.ds(s * RPT, RPT)], m_out.at[c, pl.ds(s * RPT, RPT)])


def _pre_body(deg_ref, x_ref, w_ref, g_ref, dinv_ref):
    deg = deg_ref[0] + deg_ref[1] + 1.0
    dinvb = lax.rsqrt(deg)
    hw = jnp.dot(x_ref[...], w_ref[...], preferred_element_type=jnp.float32)
    g_ref[...] = dinvb * hw
    dinv_ref[...] = dinvb


_pre = pl.pallas_call(
    _pre_body,
    grid=(NB,),
    in_specs=[
        pl.BlockSpec((2, R, H), lambda i: (0, i, 0)),
        pl.BlockSpec((R, H), lambda i: (i, 0)),
        pl.BlockSpec((H, H), lambda i: (0, 0)),
    ],
    out_specs=[
        pl.BlockSpec((R, H), lambda i: (i, 0)),
        pl.BlockSpec((R, H), lambda i: (i, 0)),
    ],
    out_shape=[
        jax.ShapeDtypeStruct((NPAD, H), jnp.float32),
        jax.ShapeDtypeStruct((NPAD, H), jnp.float32),
    ],
)


def _mid_body(m_ref, g_ref, dinv_ref, b_ref, w_ref, o_ref):
    t = (m_ref[0] + m_ref[1] + g_ref[...]) * dinv_ref[...] + b_ref[...]
    h = jnp.maximum(t, 0.0)
    o_ref[...] = dinv_ref[...] * jnp.dot(
        h, w_ref[...], preferred_element_type=jnp.float32)


_mid = pl.pallas_call(
    _mid_body,
    grid=(NB,),
    in_specs=[
        pl.BlockSpec((2, R, H), lambda i: (0, i, 0)),
        pl.BlockSpec((R, H), lambda i: (i, 0)),
        pl.BlockSpec((R, H), lambda i: (i, 0)),
        pl.BlockSpec((1, H), lambda i: (0, 0)),
        pl.BlockSpec((H, H), lambda i: (0, 0)),
    ],
    out_specs=pl.BlockSpec((R, H), lambda i: (i, 0)),
    out_shape=jax.ShapeDtypeStruct((NPAD, H), jnp.float32),
)


def _final_body(m_ref, g_ref, dinv_ref, b_ref, bt_ref, wl_ref, bl_ref,
                o_ref, acc, cnt):
    i = pl.program_id(0)

    @pl.when(i == 0)
    def _():
        acc[...] = jnp.zeros_like(acc)
        cnt[...] = jnp.zeros_like(cnt)

    h = (m_ref[0] + m_ref[1] + g_ref[...]) * dinv_ref[...] + b_ref[...]
    bt = bt_ref[0]
    P = (lax.broadcasted_iota(jnp.int32, (G, R), 0) == bt
         ).astype(jnp.float32)
    acc[...] += jnp.dot(P, h, preferred_element_type=jnp.float32)
    cnt[...] += jnp.dot(P, jnp.ones((R, H), jnp.float32),
                        preferred_element_type=jnp.float32)

    @pl.when(i == NB - 1)
    def _():
        pooled = acc[...] / jnp.maximum(cnt[...], 1.0)
        o_ref[...] = jnp.dot(pooled, wl_ref[...],
                             preferred_element_type=jnp.float32) + bl_ref[...]


_final = pl.pallas_call(
    _final_body,
    grid=(NB,),
    in_specs=[
        pl.BlockSpec((2, R, H), lambda i: (0, i, 0)),
        pl.BlockSpec((R, H), lambda i: (i, 0)),
        pl.BlockSpec((R, H), lambda i: (i, 0)),
        pl.BlockSpec((1, H), lambda i: (0, 0)),
        pl.BlockSpec((1, 1, R), lambda i: (i, 0, 0)),
        pl.BlockSpec((H, H), lambda i: (0, 0)),
        pl.BlockSpec((1, H), lambda i: (0, 0)),
    ],
    out_specs=pl.BlockSpec((G, H), lambda i: (0, 0)),
    out_shape=jax.ShapeDtypeStruct((G, H), jnp.float32),
    scratch_shapes=[
        pltpu.VMEM((G, H), jnp.float32),
        pltpu.VMEM((G, H), jnp.float32),
    ],
)


def kernel(x, edge_index, batch, W1, b1, W2, b2, W3, b3, Wl, bl):
    src = edge_index[0]
    dst = edge_index[1]
    pad_e = EPAD - E
    src_r = jnp.concatenate(
        [src, jnp.zeros((pad_e,), jnp.int32)]).reshape(NCH, K)
    dst_r = jnp.concatenate(
        [dst, jnp.full((pad_e,), N, jnp.int32)]).reshape(NCH, K)

    x_pad = jnp.pad(x, ((0, NPAD - N), (0, 0)))
    batch_p = jnp.pad(batch, (0, NPAD - N),
                      constant_values=G).reshape(NB, 1, R)
    Wl_p = jnp.pad(Wl, ((0, 0), (0, H - C)))
    bl_p = jnp.pad(bl, (0, H - C)).reshape(1, H)

    ones128 = jnp.ones((K, H), jnp.float32)
    z128 = jnp.zeros((RPT, H), jnp.float32)

    deg_par = _deg_kernel(dst_r, ones128, z128)
    g1, dinvb = _pre(deg_par, x_pad, W1)
    m1 = _mp_kernel(g1, src_r, dst_r, z128)
    g2 = _mid(m1, g1, dinvb, b1.reshape(1, H), W2)
    m2 = _mp_kernel(g2, src_r, dst_r, z128)
    g3 = _mid(m2, g2, dinvb, b2.reshape(1, H), W3)
    m3 = _mp_kernel(g3, src_r, dst_r, z128)
    out = _final(m3, g3, dinvb, b3.reshape(1, H), batch_p, Wl_p, bl_p)
    return out[:, :C]


# speedup vs baseline: 1.6627x; 1.0611x over previous
import functools
import jax
import jax.numpy as jnp
from jax import lax
from jax.experimental import pallas as pl
from jax.experimental.pallas import tpu as pltpu
from jax.experimental.pallas import tpu_sc as plsc

N = 10000
E = 320000
H = 128
G = 64
C = 10

NC = 2
NS = 16
NW = NC * NS
K = 128
CH = 80
NCH = NW * CH
EPAD = NCH * K
CHA = 144
CHB = 160 - CHA

NPAD = 10240
RPT = NPAD // NS
HRPT = RPT // 2
NB = 8
R = NPAD // NB

_mesh = plsc.VectorSubcoreMesh(core_axis_name="c", subcore_axis_name="s")


@functools.partial(
    pl.kernel,
    out_type=jax.ShapeDtypeStruct((NC, NPAD, H), jnp.float32),
    mesh=_mesh,
    scratch_types=[
        pltpu.VMEM((CH, K), jnp.int32),
        pltpu.VMEM((K, H), jnp.float32),
        pltpu.VMEM_SHARED((NPAD, H), jnp.float32),
    ],
)
def _deg_kernel(dst_hbm, ones_hbm, z_hbm, deg_out, dst_v, ones_v, acc):
    c = lax.axis_index("c")
    s = lax.axis_index("s")
    w = c * NS + s
    pltpu.sync_copy(dst_hbm.at[pl.ds(w * CH, CH)], dst_v)
    pltpu.sync_copy(ones_hbm, ones_v)
    pltpu.sync_copy(z_hbm, acc.at[pl.ds(s * RPT, RPT)])
    plsc.subcore_barrier()

    def body(j, carry):
        pltpu.sync_copy(ones_v, acc.at[dst_v.at[j]], add=True)
        return carry

    lax.fori_loop(0, CH, body, 0)
    plsc.subcore_barrier()
    pltpu.sync_copy(acc.at[pl.ds(s * RPT, RPT)], deg_out.at[c, pl.ds(s * RPT, RPT)])


W = 16


@functools.partial(
    pl.kernel,
    out_type=jax.ShapeDtypeStruct((NC, NPAD, H), jnp.float32),
    mesh=_mesh,
    scratch_types=[
        pltpu.VMEM((2, W, K), jnp.int32),
        pltpu.VMEM((2, W, K), jnp.int32),
        pltpu.VMEM((2, K, H), jnp.float32),
        pltpu.SemaphoreType.DMA,
        pltpu.SemaphoreType.DMA,
        pltpu.VMEM_SHARED((NPAD, H), jnp.float32),
    ],
)
def _mp_kernel(g_hbm, src_hbm, dst_hbm, z_hbm, m_out,
               src_w, dst_w, rows, sem0, sem1, acc):
    c = lax.axis_index("c")
    s = lax.axis_index("s")
    sems = (sem0, sem1)
    ch = jnp.where(c == 0, CHA, CHB)
    nwin = ch // W
    base = jnp.where(c == 0, s * CHA, NS * CHA + s * CHB)
    pltpu.sync_copy(src_hbm.at[pl.ds(base, W)], src_w.at[0])
    pltpu.sync_copy(dst_hbm.at[pl.ds(base, W)], dst_w.at[0])
    pltpu.sync_copy(z_hbm, acc.at[pl.ds(s * RPT, RPT)])
    plsc.subcore_barrier()

    @pl.when(ch > 0)
    def _():
        pltpu.async_copy(g_hbm.at[src_w.at[0, 0]], rows.at[0], sems[0])

    def win(v, carry):
        v2 = lax.rem(v, 2)
        v2n = lax.rem(v + 1, 2)

        @pl.when(v + 1 < nwin)
        def _():
            pltpu.sync_copy(src_hbm.at[pl.ds(base + (v + 1) * W, W)],
                            src_w.at[v2n])
            pltpu.sync_copy(dst_hbm.at[pl.ds(base + (v + 1) * W, W)],
                            dst_w.at[v2n])

        for t in range(W):
            j = v * W + t
            b, bn = t % 2, (t + 1) % 2
            pltpu.make_async_copy(
                g_hbm.at[src_w.at[v2, t]], rows.at[b], sems[b]).wait()
            nxt = (src_w.at[v2, t + 1] if t + 1 < W
                   else src_w.at[v2n, 0])

            @pl.when(j + 1 < ch)
            def _():
                pltpu.async_copy(g_hbm.at[nxt], rows.at[bn], sems[bn])

            pltpu.sync_copy(rows.at[b], acc.at[dst_w.at[v2, t]], add=True)
        return carry

    lax.fori_loop(0, nwin, win, 0)
    plsc.subcore_barrier()
    pltpu.sync_copy(acc.at[pl.ds(s * RPT, RPT)], m_out.at[c, pl.ds(s * RPT, RPT)])


def _pre_body(deg_ref, x_ref, w_ref, g_ref, dinv_ref):
    deg = deg_ref[0] + deg_ref[1] + 1.0
    dinvb = lax.rsqrt(deg)
    hw = jnp.dot(x_ref[...], w_ref[...], preferred_element_type=jnp.float32)
    g_ref[...] = dinvb * hw
    dinv_ref[...] = dinvb


_pre = pl.pallas_call(
    _pre_body,
    grid=(NB,),
    in_specs=[
        pl.BlockSpec((2, R, H), lambda i: (0, i, 0)),
        pl.BlockSpec((R, H), lambda i: (i, 0)),
        pl.BlockSpec((H, H), lambda i: (0, 0)),
    ],
    out_specs=[
        pl.BlockSpec((R, H), lambda i: (i, 0)),
        pl.BlockSpec((R, H), lambda i: (i, 0)),
    ],
    out_shape=[
        jax.ShapeDtypeStruct((NPAD, H), jnp.float32),
        jax.ShapeDtypeStruct((NPAD, H), jnp.float32),
    ],
)


def _mid_body(m_ref, g_ref, dinv_ref, b_ref, w_ref, o_ref):
    t = (m_ref[0] + m_ref[1] + g_ref[...]) * dinv_ref[...] + b_ref[...]
    h = jnp.maximum(t, 0.0)
    o_ref[...] = dinv_ref[...] * jnp.dot(
        h, w_ref[...], preferred_element_type=jnp.float32)


_mid = pl.pallas_call(
    _mid_body,
    grid=(NB,),
    in_specs=[
        pl.BlockSpec((2, R, H), lambda i: (0, i, 0)),
        pl.BlockSpec((R, H), lambda i: (i, 0)),
        pl.BlockSpec((R, H), lambda i: (i, 0)),
        pl.BlockSpec((1, H), lambda i: (0, 0)),
        pl.BlockSpec((H, H), lambda i: (0, 0)),
    ],
    out_specs=pl.BlockSpec((R, H), lambda i: (i, 0)),
    out_shape=jax.ShapeDtypeStruct((NPAD, H), jnp.float32),
)


def _final_body(m_ref, g_ref, dinv_ref, b_ref, bt_ref, wl_ref, bl_ref,
                o_ref, acc, cnt):
    i = pl.program_id(0)

    @pl.when(i == 0)
    def _():
        acc[...] = jnp.zeros_like(acc)
        cnt[...] = jnp.zeros_like(cnt)

    h = (m_ref[0] + m_ref[1] + g_ref[...]) * dinv_ref[...] + b_ref[...]
    bt = bt_ref[0]
    P = (lax.broadcasted_iota(jnp.int32, (G, R), 0) == bt
         ).astype(jnp.float32)
    acc[...] += jnp.dot(P, h, preferred_element_type=jnp.float32)
    cnt[...] += jnp.dot(P, jnp.ones((R, H), jnp.float32),
                        preferred_element_type=jnp.float32)

    @pl.when(i == NB - 1)
    def _():
        pooled = acc[...] / jnp.maximum(cnt[...], 1.0)
        o_ref[...] = jnp.dot(pooled, wl_ref[...],
                             preferred_element_type=jnp.float32) + bl_ref[...]


_final = pl.pallas_call(
    _final_body,
    grid=(NB,),
    in_specs=[
        pl.BlockSpec((2, R, H), lambda i: (0, i, 0)),
        pl.BlockSpec((R, H), lambda i: (i, 0)),
        pl.BlockSpec((R, H), lambda i: (i, 0)),
        pl.BlockSpec((1, H), lambda i: (0, 0)),
        pl.BlockSpec((1, 1, R), lambda i: (i, 0, 0)),
        pl.BlockSpec((H, H), lambda i: (0, 0)),
        pl.BlockSpec((1, H), lambda i: (0, 0)),
    ],
    out_specs=pl.BlockSpec((G, H), lambda i: (0, 0)),
    out_shape=jax.ShapeDtypeStruct((G, H), jnp.float32),
    scratch_shapes=[
        pltpu.VMEM((G, H), jnp.float32),
        pltpu.VMEM((G, H), jnp.float32),
    ],
)


def kernel(x, edge_index, batch, W1, b1, W2, b2, W3, b3, Wl, bl):
    src = edge_index[0]
    dst = edge_index[1]
    pad_e = EPAD - E
    src_r = jnp.concatenate(
        [src, jnp.zeros((pad_e,), jnp.int32)]).reshape(NCH, K)
    dst_r = jnp.concatenate(
        [dst, jnp.full((pad_e,), N, jnp.int32)]).reshape(NCH, K)

    x_pad = jnp.pad(x, ((0, NPAD - N), (0, 0)))
    batch_p = jnp.pad(batch, (0, NPAD - N),
                      constant_values=G).reshape(NB, 1, R)
    Wl_p = jnp.pad(Wl, ((0, 0), (0, H - C)))
    bl_p = jnp.pad(bl, (0, H - C)).reshape(1, H)

    ones128 = jnp.ones((K, H), jnp.float32)
    z128 = jnp.zeros((RPT, H), jnp.float32)

    deg_par = _deg_kernel(dst_r, ones128, z128)
    g1, dinvb = _pre(deg_par, x_pad, W1)
    m1 = _mp_kernel(g1, src_r, dst_r, z128)
    g2 = _mid(m1, g1, dinvb, b1.reshape(1, H), W2)
    m2 = _mp_kernel(g2, src_r, dst_r, z128)
    g3 = _mid(m2, g2, dinvb, b2.reshape(1, H), W3)
    m3 = _mp_kernel(g3, src_r, dst_r, z128)
    out = _final(m3, g3, dinvb, b3.reshape(1, H), batch_p, Wl_p, bl_p)
    return out[:, :C]
